# baseline (speedup 1.0000x reference)
import jax
import jax.numpy as jnp
from jax import lax
from jax.experimental import pallas as pl
from jax.experimental.pallas import tpu as pltpu
from jax.experimental.pallas import tpu_sc as plsc

D = 14336
NC, NS = 2, 16
NW = NC * NS
B = 512
SC_B = 256
TC_B = B - SC_B
BPW = SC_B // NW
CH = 2
NB = 2
NCHUNK = BPW // CH


def _sc_body(idx_hbm, table_hbm, out_hbm, idx_v, buf0, buf1, g0, g1, w0, w1):
    wid = lax.axis_index("s") * NC + lax.axis_index("c")
    base = wid * BPW
    pltpu.sync_copy(idx_hbm.at[wid], idx_v)
    bufs = (buf0, buf1)
    gsems = (g0, g1)
    wsems = (w0, w1)

    def gather(j, b):
        return pltpu.make_async_copy(
            table_hbm.at[idx_v.at[j]], bufs[b], gsems[b])

    def write(j, b):
        return pltpu.make_async_copy(
            bufs[b], out_hbm.at[pl.ds(base + j * CH, CH)], wsems[b])

    for b in range(NB):
        gather(b, b).start()

    def step(t, carry):
        for b in range(NB):
            j = t * NB + b
            gather(j, b).wait()
            write(j, b).start()
            write(j, b).wait()
            gather(j + NB, b).start()
        return carry

    lax.fori_loop(0, NCHUNK // NB - 1, step, 0)
    for b in range(NB):
        j = NCHUNK - NB + b
        gather(j, b).wait()
        write(j, b).start()
    for b in range(NB):
        write(NCHUNK - NB + b, b).wait()


_sc_call = pl.kernel(
    _sc_body,
    out_type=jax.ShapeDtypeStruct((SC_B, D), jnp.float32),
    mesh=plsc.VectorSubcoreMesh(core_axis_name="c", subcore_axis_name="s"),
    scratch_types=(
        [pltpu.VMEM((NCHUNK, CH), jnp.int32)]
        + [pltpu.VMEM((CH, D), jnp.float32)] * NB
        + [pltpu.SemaphoreType.DMA] * (2 * NB)
    ),
)


def _tc_body(idx_ref, row_ref, out_ref):
    out_ref[...] = row_ref[...]


def _tc_call(idx, table):
    out = pl.pallas_call(
        _tc_body,
        grid_spec=pltpu.PrefetchScalarGridSpec(
            num_scalar_prefetch=1,
            grid=(TC_B,),
            in_specs=[pl.BlockSpec((1, 1, D),
                                   lambda i, idx_ref: (idx_ref[i], 0, 0))],
            out_specs=pl.BlockSpec((1, 1, D), lambda i, idx_ref: (i, 0, 0)),
        ),
        out_shape=jax.ShapeDtypeStruct((TC_B, 1, D), jnp.float32),
    )(idx, table.reshape(table.shape[0], 1, D))
    return out.reshape(TC_B, D)


def kernel(prefix, embedding_table):
    bsz, seq = prefix.shape
    flat = prefix.astype(jnp.int32).reshape(B)
    sc_idx = flat[:SC_B].reshape(NW, NCHUNK, CH)
    tc_idx = flat[SC_B:]
    sc_out = _sc_call(sc_idx, embedding_table)
    tc_out = _tc_call(tc_idx, embedding_table)
    out = jnp.concatenate([sc_out, tc_out], axis=0)
    return out.reshape(bsz, seq, D)
